# Initial kernel scaffold; baseline (speedup 1.0000x reference)
import jax
import jax.numpy as jnp
from jax.experimental import pallas as pl


def kernel(species12, distances, cutoff_radii):
    raise NotImplementedError("write your pallas kernel here")


# speedup vs baseline: 300.9286x; 300.9286x over previous
import functools

import jax
import jax.numpy as jnp
from jax import lax
from jax.experimental import pallas as pl
from jax.experimental.pallas import tpu as pltpu
from jax.experimental.pallas import tpu_sc as plsc

_A1 = 0.4
_A2 = 4.4
_NT = 4
_P = 6400000
_NC = 2
_NS = 16
_NW = _NC * _NS
_L = 16
_PER_W = _P // _NW
_C = 10000
_G = _PER_W // _C


def _body(species_hbm, dist_hbm, cr_hbm, out_hbm, table_v, s0_v, s1_v, d_v, o_v):
    wid = lax.axis_index("s") * _NC + lax.axis_index("c")

    pltpu.sync_copy(cr_hbm, table_v)
    t = table_v[...] * _A1 + _A2
    t2 = t * t
    table_v[...] = t2 * t2 * t2

    base_w = wid * _PER_W

    def chunk(g, carry):
        base = pl.multiple_of(base_w + g * _C, 8)
        pltpu.sync_copy(species_hbm.at[pl.ds(base, _C)], s0_v)
        pltpu.sync_copy(species_hbm.at[pl.ds(base + _P, _C)], s1_v)
        pltpu.sync_copy(dist_hbm.at[pl.ds(base, _C)], d_v)

        def inner(i, c):
            sl = pl.ds(i * _L, _L)
            idx = s0_v[sl] * _NT + s1_v[sl]
            damp = plsc.load_gather(table_v, [idx])
            d = d_v[sl]
            d2 = d * d
            o_v[sl] = d2 * d2 * d2 + damp
            return c

        lax.fori_loop(0, _C // _L, inner, 0, unroll=4)
        pltpu.sync_copy(o_v, out_hbm.at[pl.ds(base, _C)])
        return carry

    lax.fori_loop(0, _G, chunk, 0)


_damp = functools.partial(
    pl.kernel,
    out_type=jax.ShapeDtypeStruct((_P,), jnp.float32),
    mesh=plsc.VectorSubcoreMesh(core_axis_name="c", subcore_axis_name="s"),
    scratch_types=[
        pltpu.VMEM((16,), jnp.float32),
        pltpu.VMEM((_C,), jnp.int32),
        pltpu.VMEM((_C,), jnp.int32),
        pltpu.VMEM((_C,), jnp.float32),
        pltpu.VMEM((_C,), jnp.float32),
    ],
    compiler_params=pltpu.CompilerParams(needs_layout_passes=False),
)(_body)


@jax.jit
def kernel(species12, distances, cutoff_radii):
    return _damp(species12.reshape(-1), distances, cutoff_radii.reshape(-1))


# speedup vs baseline: 392.7151x; 1.3050x over previous
import functools

import jax
import jax.numpy as jnp
from jax import lax
from jax.experimental import pallas as pl
from jax.experimental.pallas import tpu as pltpu
from jax.experimental.pallas import tpu_sc as plsc

_A1 = 0.4
_A2 = 4.4
_NT = 4
_P = 6400000
_NC = 2
_NS = 16
_NW = _NC * _NS
_L = 16
_PER_W = _P // _NW
_C = 10000
_G = _PER_W // _C


def _body(species_hbm, dist_hbm, cr_hbm, out_hbm, table_v,
          s0a, s0b, s1a, s1b, da, db, oa, ob,
          in_sem0, in_sem1, out_sem0, out_sem1):
    wid = lax.axis_index("s") * _NC + lax.axis_index("c")

    pltpu.sync_copy(cr_hbm, table_v)
    t = table_v[...] * _A1 + _A2
    t2 = t * t
    table_v[...] = t2 * t2 * t2

    base_w = wid * _PER_W
    s0_v = (s0a, s0b)
    s1_v = (s1a, s1b)
    d_v = (da, db)
    o_v = (oa, ob)
    in_sems = (in_sem0, in_sem1)
    out_sems = (out_sem0, out_sem1)
    in_descs = [None, None]
    out_descs = [None, None]

    def start_in(g, b):
        base = pl.multiple_of(base_w + g * _C, 8)
        in_descs[b] = (
            pltpu.async_copy(species_hbm.at[pl.ds(base, _C)], s0_v[b],
                             in_sems[b]),
            pltpu.async_copy(species_hbm.at[pl.ds(base + _P, _C)], s1_v[b],
                             in_sems[b]),
            pltpu.async_copy(dist_hbm.at[pl.ds(base, _C)], d_v[b],
                             in_sems[b]),
        )

    start_in(0, 0)
    for g in range(_G):
        b = g & 1
        if g + 1 < _G:
            start_in(g + 1, 1 - b)
        for dsc in in_descs[b]:
            dsc.wait()
        if out_descs[b] is not None:
            out_descs[b].wait()

        def inner(i, c, b=b):
            sl = pl.ds(i * _L, _L)
            idx = s0_v[b][sl] * _NT + s1_v[b][sl]
            damp = plsc.load_gather(table_v, [idx])
            d = d_v[b][sl]
            d2 = d * d
            o_v[b][sl] = d2 * d2 * d2 + damp
            return c

        lax.fori_loop(0, _C // _L, inner, 0, unroll=8)
        base = pl.multiple_of(base_w + g * _C, 8)
        out_descs[b] = pltpu.async_copy(o_v[b],
                                        out_hbm.at[pl.ds(base, _C)],
                                        out_sems[b])
    for b in (0, 1):
        if out_descs[b] is not None:
            out_descs[b].wait()


_damp = functools.partial(
    pl.kernel,
    out_type=jax.ShapeDtypeStruct((_P,), jnp.float32),
    mesh=plsc.VectorSubcoreMesh(core_axis_name="c", subcore_axis_name="s"),
    scratch_types=[
        pltpu.VMEM((16,), jnp.float32),
        pltpu.VMEM((_C,), jnp.int32),
        pltpu.VMEM((_C,), jnp.int32),
        pltpu.VMEM((_C,), jnp.int32),
        pltpu.VMEM((_C,), jnp.int32),
        pltpu.VMEM((_C,), jnp.float32),
        pltpu.VMEM((_C,), jnp.float32),
        pltpu.VMEM((_C,), jnp.float32),
        pltpu.VMEM((_C,), jnp.float32),
        pltpu.SemaphoreType.DMA,
        pltpu.SemaphoreType.DMA,
        pltpu.SemaphoreType.DMA,
        pltpu.SemaphoreType.DMA,
    ],
    compiler_params=pltpu.CompilerParams(needs_layout_passes=False),
)(_body)


@jax.jit
def kernel(species12, distances, cutoff_radii):
    return _damp(species12.reshape(-1), distances, cutoff_radii.reshape(-1))


# speedup vs baseline: 762.3386x; 1.9412x over previous
import functools

import jax
import jax.numpy as jnp
from jax import lax
from jax.experimental import pallas as pl
from jax.experimental.pallas import tpu as pltpu
from jax.experimental.pallas import tpu_sc as plsc

_A1 = 0.4
_A2 = 4.4
_NT = 4
_P = 6400000
_NC = 2
_NS = 16
_NW = _NC * _NS
_L = 16
_PER_W = _P // _NW
_C = 10000
_G = _PER_W // _C


def _body(species_hbm, dist_hbm, cr_hbm, out_hbm, table_v,
          s0a, s0b, s1a, s1b, da, db, oa, ob,
          in_sem0, in_sem1, out_sem0, out_sem1):
    wid = lax.axis_index("s") * _NC + lax.axis_index("c")

    pltpu.sync_copy(cr_hbm, table_v)
    t = table_v[...] * _A1 + _A2
    t2 = t * t
    table_v[...] = t2 * t2 * t2

    base_w = wid * _PER_W
    s0_v = (s0a, s0b)
    s1_v = (s1a, s1b)
    d_v = (da, db)
    o_v = (oa, ob)
    in_sems = (in_sem0, in_sem1)
    out_sems = (out_sem0, out_sem1)
    in_descs = [None, None]
    out_descs = [None, None]

    def start_in(g, b):
        base = pl.multiple_of(base_w + g * _C, 8)
        in_descs[b] = (
            pltpu.async_copy(species_hbm.at[pl.ds(base, _C)], s0_v[b],
                             in_sems[b]),
            pltpu.async_copy(species_hbm.at[pl.ds(base + _P, _C)], s1_v[b],
                             in_sems[b]),
            pltpu.async_copy(dist_hbm.at[pl.ds(base, _C)], d_v[b],
                             in_sems[b]),
        )

    start_in(0, 0)
    for g in range(_G):
        b = g & 1
        if g + 1 < _G:
            start_in(g + 1, 1 - b)
        for dsc in in_descs[b]:
            dsc.wait()
        if out_descs[b] is not None:
            out_descs[b].wait()

        @plsc.parallel_loop(0, _C, _L, unroll=8)
        def inner(i, b=b):
            sl = pl.ds(i, _L)
            idx = s0_v[b][sl] * _NT + s1_v[b][sl]
            damp = plsc.load_gather(table_v, [idx])
            d = d_v[b][sl]
            d2 = d * d
            o_v[b][sl] = d2 * d2 * d2 + damp
        base = pl.multiple_of(base_w + g * _C, 8)
        out_descs[b] = pltpu.async_copy(o_v[b],
                                        out_hbm.at[pl.ds(base, _C)],
                                        out_sems[b])
    for b in (0, 1):
        if out_descs[b] is not None:
            out_descs[b].wait()


_damp = functools.partial(
    pl.kernel,
    out_type=jax.ShapeDtypeStruct((_P,), jnp.float32),
    mesh=plsc.VectorSubcoreMesh(core_axis_name="c", subcore_axis_name="s"),
    scratch_types=[
        pltpu.VMEM((16,), jnp.float32),
        pltpu.VMEM((_C,), jnp.int32),
        pltpu.VMEM((_C,), jnp.int32),
        pltpu.VMEM((_C,), jnp.int32),
        pltpu.VMEM((_C,), jnp.int32),
        pltpu.VMEM((_C,), jnp.float32),
        pltpu.VMEM((_C,), jnp.float32),
        pltpu.VMEM((_C,), jnp.float32),
        pltpu.VMEM((_C,), jnp.float32),
        pltpu.SemaphoreType.DMA,
        pltpu.SemaphoreType.DMA,
        pltpu.SemaphoreType.DMA,
        pltpu.SemaphoreType.DMA,
    ],
    compiler_params=pltpu.CompilerParams(needs_layout_passes=False),
)(_body)


@jax.jit
def kernel(species12, distances, cutoff_radii):
    return _damp(species12.reshape(-1), distances, cutoff_radii.reshape(-1))
